# baseline (speedup 1.0000x reference)
import functools

import numpy as np
import jax
import jax.numpy as jnp
from jax.experimental import pallas as pl
from jax.experimental.pallas import tpu as pltpu

_B = 16
_N = 8192
_S = 64
_C = 128
_NPTS = 1024
_K = 32
_TOP = 4


_ROT = [[13, 15, 26, 6], [17, 29, 16, 24]]


def _rotl(x, r):
    return ((x << np.uint32(r)) | (x >> np.uint32(32 - r))).astype(np.uint32)


def _tf2x32(k0, k1, x0, x1):
    k0 = np.asarray(k0, np.uint32)
    k1 = np.asarray(k1, np.uint32)
    ks = [k0, k1, (k0 ^ k1 ^ np.uint32(0x1BD11BDA)).astype(np.uint32)]
    x = [(np.asarray(x0, np.uint32) + ks[0]).astype(np.uint32),
         (np.asarray(x1, np.uint32) + ks[1]).astype(np.uint32)]
    for i in range(5):
        for r in _ROT[i % 2]:
            x[0] = (x[0] + x[1]).astype(np.uint32)
            x[1] = _rotl(x[1], r)
            x[1] = (x[1] ^ x[0]).astype(np.uint32)
        x[0] = (x[0] + ks[(i + 1) % 3]).astype(np.uint32)
        x[1] = (x[1] + ks[(i + 2) % 3] + np.uint32(i + 1)).astype(np.uint32)
    return x[0], x[1]


def _rng_tables():
    seed = np.array([0, 42], np.uint32)
    bs = np.arange(_B, dtype=np.uint32)
    k0, k1 = _tf2x32(seed[0], seed[1], np.zeros(_B, np.uint32), bs)
    hb = np.zeros((_NPTS, _B), np.uint32)
    lb = np.zeros((_NPTS, _B), np.uint32)
    z = np.zeros(_B, np.uint32)
    for i in range(_NPTS):
        a0, a1 = _tf2x32(k0, k1, z, z)
        b0, b1 = _tf2x32(k0, k1, z, z + np.uint32(1))
        c0, c1 = _tf2x32(b0, b1, z, z)
        d0, d1 = _tf2x32(b0, b1, z, z + np.uint32(1))
        e0, e1 = _tf2x32(c0, c1, z, z)
        f0, f1 = _tf2x32(d0, d1, z, z)
        hb[i] = e0 ^ e1
        lb[i] = f0 ^ f1
        k0, k1 = a0, a1
    return hb, lb


_HB_NP, _LB_NP = _rng_tables()


def _mod(a, s):
    for k in range(17, -1, -1):
        t = s << k
        a = jnp.where(a >= t, a - t, a)
    return a


def _scan_cols(m):
    sh = 1
    while sh < m.shape[1]:
        z = jnp.zeros((m.shape[0], sh), m.dtype)
        m = m + jnp.concatenate([z, m[:, :-sh]], axis=1)
        sh *= 2
    return m


def _body(npts, x_ref, y_ref, z_ref, hb_ref, lb_ref, idx_ref, pts_ref,
          used_ref):
    x = x_ref[...]
    y = y_ref[...]
    z = z_ref[...]
    iota_s = jax.lax.broadcasted_iota(jnp.int32, (_B, _S, _C), 1)
    iota_c3 = jax.lax.broadcasted_iota(jnp.int32, (_B, _S, _C), 2)
    gidx = iota_c3 * _S + iota_s
    iota_c = jax.lax.broadcasted_iota(jnp.int32, (_B, _C), 1)
    used_ref[...] = jnp.zeros((_B, _S, _C), jnp.int32)
    inf = jnp.float32(np.inf)
    big = jnp.int32(_N)

    def step(i, cid):
        used = used_ref[...]
        cid3 = cid[:, :, None]
        mask0 = (used == cid3)
        colcnt0 = jnp.sum(mask0.astype(jnp.int32), axis=1)
        cnt0 = jnp.sum(colcnt0, axis=1, keepdims=True)
        umin = jnp.min(jnp.min(used, axis=1), axis=1, keepdims=True)
        cid = jnp.where(cnt0 == 0, umin, cid)
        cid3 = cid[:, :, None]
        mask = (used == cid3)
        colcnt = jnp.sum(mask.astype(jnp.int32), axis=1)
        cnt = jnp.sum(colcnt, axis=1, keepdims=True)
        span = jnp.maximum(cnt, 1)

        hbw = jnp.reshape(hb_ref[pl.ds(i, 1)], (_B, 1))
        lbw = jnp.reshape(lb_ref[pl.ds(i, 1)], (_B, 1))
        m1 = _mod(jnp.full((_B, 1), 1 << 16, jnp.int32), span)
        mult = _mod(m1 * m1, span)

        def _mod32(w):
            hi = jax.lax.shift_right_logical(w, 16)
            lo = jax.lax.bitwise_and(w, (1 << 16) - 1)
            him = _mod(hi, span)
            return _mod(him * m1 + lo, span)

        r = _mod(_mod32(hbw) * mult + _mod32(lbw), span)

        cc = _scan_cols(colcnt)
        want = r + 1
        csel = jnp.min(jnp.where(cc >= want, iota_c, _C), axis=1,
                       keepdims=True)
        prev = cc - colcnt
        pc = jnp.sum(jnp.where(iota_c == csel, prev, 0), axis=1,
                     keepdims=True)
        t = want - pc
        csel3 = csel[:, :, None]
        mcol = jnp.sum(jnp.where(iota_c3 == csel3, mask.astype(jnp.int32), 0),
                       axis=2)
        cs = _scan_cols(mcol)
        iota_r = jax.lax.broadcasted_iota(jnp.int32, (_B, _S), 1)
        row = jnp.min(jnp.where((cs == t) & (mcol > 0), iota_r, _S), axis=1,
                      keepdims=True)
        index = csel * _S + row

        sel = (gidx == index[:, :, None])
        pxv = jnp.sum(jnp.sum(jnp.where(sel, x, 0.0), axis=1), axis=1,
                      keepdims=True)
        pyv = jnp.sum(jnp.sum(jnp.where(sel, y, 0.0), axis=1), axis=1,
                      keepdims=True)
        pzv = jnp.sum(jnp.sum(jnp.where(sel, z, 0.0), axis=1), axis=1,
                      keepdims=True)

        dx = x - pxv[:, :, None]
        dy = y - pyv[:, :, None]
        dz = z - pzv[:, :, None]
        d = dx * dx + dy * dy + dz * dz

        dw = d
        tvs, tgs = [], []
        for _q in range(_TOP):
            mv = jnp.min(dw, axis=1)
            jr = jnp.min(jnp.where(dw == mv[:, None, :], iota_s, _S),
                         axis=1)
            tvs.append(mv)
            tgs.append(iota_c * _S + jr)
            dw = jnp.where(iota_s == jr[:, None, :], inf, dw)

        hp = jnp.zeros((_B, _C), jnp.int32)
        hv = tvs[0]
        hg = tgs[0]
        lastv = tvs[_TOP - 1]
        lastg = tgs[_TOP - 1]
        ids = []
        bad = jnp.zeros((), jnp.bool_)
        tv32 = None
        gid32 = None
        for _k in range(_K):
            mstar = jnp.min(hv, axis=1, keepdims=True)
            gstar = jnp.min(jnp.where(hv == mstar, hg, big), axis=1,
                            keepdims=True)
            exh = (hp >= _TOP)
            hidden_lt = exh & ((lastv < mstar) |
                               ((lastv == mstar) & (lastg < gstar)))
            bad = bad | jnp.any(hidden_lt)
            ids.append(gstar)
            cstar = jax.lax.shift_right_logical(gstar, 6)
            hit = (iota_c == cstar)
            hp = hp + hit.astype(jnp.int32)
            nv = jnp.full((_B, _C), inf, jnp.float32)
            ng = jnp.full((_B, _C), big, jnp.int32)
            for _q in range(1, _TOP):
                at = (hp == _q)
                nv = jnp.where(at, tvs[_q], nv)
                ng = jnp.where(at, tgs[_q], ng)
            hv = jnp.where(hit, nv, hv)
            hg = jnp.where(hit, ng, hg)
            tv32, gid32 = mstar, gstar

        ids_fast = jnp.concatenate(ids, axis=1)
        membt = (d < tv32[:, :, None]) | \
                ((d == tv32[:, :, None]) & (gidx <= gid32[:, :, None]))

        def _fallback(_):
            dw2 = d
            out = []
            acc = jnp.zeros((_B, _S, _C), jnp.int32)
            for _k2 in range(_K):
                mcolv = jnp.min(dw2, axis=1)
                mv2 = jnp.min(mcolv, axis=1, keepdims=True)
                g2 = jnp.min(
                    jnp.where(dw2 == mv2[:, :, None], gidx, big),
                    axis=1)
                g2 = jnp.min(g2, axis=1, keepdims=True)
                out.append(g2)
                oh = (gidx == g2[:, :, None])
                dw2 = jnp.where(oh, inf, dw2)
                acc = acc + oh.astype(jnp.int32)
            return jnp.concatenate(out, axis=1), acc

        def _fastpath(_):
            return ids_fast, membt.astype(jnp.int32)

        ids32, member = jax.lax.cond(bad, _fallback, _fastpath, 0)

        used = used + member + jnp.where(sel, 100, 0)
        used_ref[...] = used

        idx_ref[pl.ds(i, 1)] = jnp.reshape(ids32, (1, _B, _K))
        pts_ref[pl.ds(i, 1)] = jnp.reshape(
            jnp.concatenate([pxv, pyv, pzv], axis=1), (1, _B, 3))
        return cid

    jax.lax.fori_loop(0, npts, step, jnp.zeros((_B, 1), jnp.int32))


@functools.partial(jax.jit, static_argnums=(1, 2))
def _run(xyz1, npts, interpret):
    def grid(a):
        return jnp.transpose(jnp.reshape(a, (_B, _C, _S)), (0, 2, 1))

    x = grid(xyz1[:, :, 0])
    y = grid(xyz1[:, :, 1])
    z = grid(xyz1[:, :, 2])
    hb = jnp.asarray(_HB_NP[:npts].astype(np.int32)).reshape(npts, _B, 1)
    lb = jnp.asarray(_LB_NP[:npts].astype(np.int32)).reshape(npts, _B, 1)
    idx, pts = pl.pallas_call(
        functools.partial(_body, npts),
        out_shape=[
            jax.ShapeDtypeStruct((npts, _B, _K), jnp.int32),
            jax.ShapeDtypeStruct((npts, _B, 3), jnp.float32),
        ],
        scratch_shapes=[pltpu.VMEM((_B, _S, _C), jnp.int32)],
        interpret=interpret,
    )(x, y, z, hb, lb)
    idx = jnp.transpose(idx, (1, 0, 2))[..., None]
    pts = jnp.transpose(pts, (1, 0, 2))
    return idx, pts


def kernel(xyz1):
    return _run(jax.lax.stop_gradient(xyz1), _NPTS, False)


# speedup vs baseline: 1.1997x; 1.1997x over previous
import functools

import numpy as np
import jax
import jax.numpy as jnp
from jax.experimental import pallas as pl
from jax.experimental.pallas import tpu as pltpu

_B = 16
_N = 8192
_NPTS = 1024
_K = 32


_ROT = [[13, 15, 26, 6], [17, 29, 16, 24]]


def _rotl(x, r):
    return ((x << np.uint32(r)) | (x >> np.uint32(32 - r))).astype(np.uint32)


def _tf2x32(k0, k1, x0, x1):
    k0 = np.asarray(k0, np.uint32)
    k1 = np.asarray(k1, np.uint32)
    ks = [k0, k1, (k0 ^ k1 ^ np.uint32(0x1BD11BDA)).astype(np.uint32)]
    x = [(np.asarray(x0, np.uint32) + ks[0]).astype(np.uint32),
         (np.asarray(x1, np.uint32) + ks[1]).astype(np.uint32)]
    for i in range(5):
        for r in _ROT[i % 2]:
            x[0] = (x[0] + x[1]).astype(np.uint32)
            x[1] = _rotl(x[1], r)
            x[1] = (x[1] ^ x[0]).astype(np.uint32)
        x[0] = (x[0] + ks[(i + 1) % 3]).astype(np.uint32)
        x[1] = (x[1] + ks[(i + 2) % 3] + np.uint32(i + 1)).astype(np.uint32)
    return x[0], x[1]


def _rng_tables():
    seed = np.array([0, 42], np.uint32)
    bs = np.arange(_B, dtype=np.uint32)
    k0, k1 = _tf2x32(seed[0], seed[1], np.zeros(_B, np.uint32), bs)
    hb = np.zeros((_NPTS, _B), np.uint32)
    lb = np.zeros((_NPTS, _B), np.uint32)
    z = np.zeros(_B, np.uint32)
    for i in range(_NPTS):
        a0, a1 = _tf2x32(k0, k1, z, z)
        b0, b1 = _tf2x32(k0, k1, z, z + np.uint32(1))
        c0, c1 = _tf2x32(b0, b1, z, z)
        d0, d1 = _tf2x32(b0, b1, z, z + np.uint32(1))
        e0, e1 = _tf2x32(c0, c1, z, z)
        f0, f1 = _tf2x32(d0, d1, z, z)
        hb[i] = e0 ^ e1
        lb[i] = f0 ^ f1
        k0, k1 = a0, a1
    return hb, lb


_HB_NP, _LB_NP = _rng_tables()


def _mod(a, s):
    for k in range(17, -1, -1):
        t = s << k
        a = jnp.where(a >= t, a - t, a)
    return a


def _lane_cumsum(m):
    sh = 1
    while sh < _N:
        z = jnp.zeros((_B, sh), m.dtype)
        m = m + jnp.concatenate([z, m[:, :-sh]], axis=1)
        sh *= 2
    return m


def _body(npts, x_ref, y_ref, z_ref, hb_ref, lb_ref, idx_ref, pts_ref,
          used_ref):
    x = x_ref[...]
    y = y_ref[...]
    z = z_ref[...]
    iota = jax.lax.broadcasted_iota(jnp.int32, (_B, _N), 1)
    used_ref[...] = jnp.zeros((_B, _N), jnp.int32)
    inf = jnp.float32(np.inf)

    def step(i, cid):
        used = used_ref[...]
        mask0 = (used == cid)
        cnt0 = jnp.sum(mask0.astype(jnp.int32), axis=1, keepdims=True)
        umin = jnp.min(used, axis=1, keepdims=True)
        cid = jnp.where(cnt0 == 0, umin, cid)
        mask = (used == cid)
        cnt = jnp.sum(mask.astype(jnp.int32), axis=1, keepdims=True)
        span = jnp.maximum(cnt, 1)

        hbw = jnp.reshape(hb_ref[pl.ds(i, 1)], (_B, 1))
        lbw = jnp.reshape(lb_ref[pl.ds(i, 1)], (_B, 1))
        m1 = _mod(jnp.full((_B, 1), 1 << 16, jnp.int32), span)
        mult = _mod(m1 * m1, span)

        def _mod32(w):
            hi = jax.lax.shift_right_logical(w, 16)
            lo = jax.lax.bitwise_and(w, (1 << 16) - 1)
            him = _mod(hi, span)
            return _mod(him * m1 + lo, span)

        r = _mod(_mod32(hbw) * mult + _mod32(lbw), span)

        csum = _lane_cumsum(mask.astype(jnp.int32))
        hit = (csum == (r + 1)) & mask
        index = jnp.min(jnp.where(hit, iota, _N), axis=1, keepdims=True)

        sel = (iota == index)
        px = jnp.sum(jnp.where(sel, x, 0.0), axis=1, keepdims=True)
        py = jnp.sum(jnp.where(sel, y, 0.0), axis=1, keepdims=True)
        pz = jnp.sum(jnp.where(sel, z, 0.0), axis=1, keepdims=True)

        dx = x - px
        dy = y - py
        dz = z - pz
        d = dx * dx + dy * dy + dz * dz

        ids = []
        dw = d
        m = None
        j = None
        for _ in range(_K):
            m = jnp.min(dw, axis=1, keepdims=True)
            j = jnp.min(jnp.where(dw == m, iota, _N), axis=1, keepdims=True)
            ids.append(j)
            dw = jnp.where(iota == j, inf, dw)

        member = (d < m) | ((d == m) & (iota <= j))
        used = used + member.astype(jnp.int32) + jnp.where(sel, 100, 0)
        used_ref[...] = used

        idx_ref[pl.ds(i, 1)] = jnp.reshape(
            jnp.concatenate(ids, axis=1), (1, _B, _K))
        pts_ref[pl.ds(i, 1)] = jnp.reshape(
            jnp.concatenate([px, py, pz], axis=1), (1, _B, 3))
        return cid

    jax.lax.fori_loop(0, npts, step, jnp.zeros((_B, 1), jnp.int32))


@functools.partial(jax.jit, static_argnums=(1, 2))
def _run(xyz1, npts, interpret):
    x = xyz1[:, :, 0]
    y = xyz1[:, :, 1]
    z = xyz1[:, :, 2]
    hb = jnp.asarray(_HB_NP[:npts].astype(np.int32)).reshape(npts, _B, 1)
    lb = jnp.asarray(_LB_NP[:npts].astype(np.int32)).reshape(npts, _B, 1)
    idx, pts = pl.pallas_call(
        functools.partial(_body, npts),
        out_shape=[
            jax.ShapeDtypeStruct((npts, _B, _K), jnp.int32),
            jax.ShapeDtypeStruct((npts, _B, 3), jnp.float32),
        ],
        scratch_shapes=[pltpu.VMEM((_B, _N), jnp.int32)],
        interpret=interpret,
    )(x, y, z, hb, lb)
    idx = jnp.transpose(idx, (1, 0, 2))[..., None]
    pts = jnp.transpose(pts, (1, 0, 2))
    return idx, pts


def kernel(xyz1):
    return _run(jax.lax.stop_gradient(xyz1), _NPTS, False)
